# Initial kernel scaffold; baseline (speedup 1.0000x reference)
import jax
import jax.numpy as jnp
from jax.experimental import pallas as pl


def kernel(x, edge_index, W_l, b_l, W_r, ln_gamma, ln_beta):
    raise NotImplementedError("write your pallas kernel here")


# speedup vs baseline: 12.1950x; 12.1950x over previous
import functools

import jax
import jax.numpy as jnp
from jax import lax
from jax.experimental import pallas as pl
from jax.experimental.pallas import tpu as pltpu
from jax.experimental.pallas import tpu_sc as plsc

N = 10000
E = 320000
D = 128

NC = 2
NS = 16
NW = NC * NS
EPW = E // NW
C = 80
NCHUNK = EPW // C
RPT = N // NS
ZR = 125
CW = 8


def _sc_aggregate_body(x_hbm, src_hbm, dst_hbm, zrows_hbm, consts_hbm,
                       part_hbm, cntp_hbm,
                       src_v, dst_v, rows0, rows1,
                       ones_v,
                       acc_sh, cnt_sh,
                       sem0, sem1):
    cid = lax.axis_index("c")
    sid = lax.axis_index("s")
    wid = cid * NS + sid

    pltpu.sync_copy(src_hbm.at[wid], src_v)
    pltpu.sync_copy(dst_hbm.at[wid], dst_v)

    pltpu.sync_copy(consts_hbm.at[pl.ds(RPT, C)], ones_v)

    pltpu.sync_copy(zrows_hbm, acc_sh.at[pl.ds(sid * RPT, RPT)])
    pltpu.sync_copy(consts_hbm.at[pl.ds(0, RPT)],
                    cnt_sh.at[pl.ds(sid * RPT, RPT)])

    plsc.subcore_barrier()

    def gather(k, buf, sem):
        return pltpu.make_async_copy(x_hbm.at[src_v.at[k]], buf, sem)

    def scatter(k, buf):
        pltpu.sync_copy(buf, acc_sh.at[dst_v.at[k]], add=True)
        pltpu.sync_copy(ones_v, cnt_sh.at[dst_v.at[k]], add=True)

    gather(0, rows0, sem0).start()

    @pl.loop(0, NCHUNK // 2)
    def _chunks(k2):
        k = k2 * 2
        gather(k + 1, rows1, sem1).start()
        gather(k, rows0, sem0).wait()
        scatter(k, rows0)
        gather(k + 2, rows0, sem0).start()
        gather(k + 1, rows1, sem1).wait()
        scatter(k + 1, rows1)

    gather(NCHUNK - 1, rows0, sem0).wait()
    scatter(NCHUNK - 1, rows0)

    plsc.subcore_barrier()

    row0 = sid * RPT
    pltpu.sync_copy(acc_sh.at[pl.ds(row0, RPT)],
                    part_hbm.at[cid, pl.ds(row0, RPT)])
    pltpu.sync_copy(cnt_sh.at[pl.ds(row0, RPT)],
                    cntp_hbm.at[cid, pl.ds(row0, RPT)])


_sc_aggregate = pl.kernel(
    _sc_aggregate_body,
    out_type=(jax.ShapeDtypeStruct((NC, N, D), jnp.float32),
              jax.ShapeDtypeStruct((NC, N, CW), jnp.float32)),
    mesh=plsc.VectorSubcoreMesh(core_axis_name="c", subcore_axis_name="s"),
    scratch_types=[
        pltpu.VMEM((NCHUNK, C), jnp.int32),
        pltpu.VMEM((NCHUNK, C), jnp.int32),
        pltpu.VMEM((C, D), jnp.float32),
        pltpu.VMEM((C, D), jnp.float32),
        pltpu.VMEM((C, CW), jnp.float32),
        pltpu.VMEM_SHARED((N, D), jnp.float32),
        pltpu.VMEM_SHARED((N, CW), jnp.float32),
        pltpu.SemaphoreType.DMA,
        pltpu.SemaphoreType.DMA,
    ],
    compiler_params=pltpu.CompilerParams(use_tc_tiling_on_sc=False),
)


ROWS_PER_BLK = 1000


def _tc_fuse_body(part_ref, cnt_ref, x_ref, wl_ref, wr_ref, b_ref,
                  g_ref, beta_ref, o_ref):
    s = part_ref[0] + part_ref[1]
    c = cnt_ref[0, :, 0:1] + cnt_ref[1, :, 0:1]
    mean = s / jnp.maximum(c, 1.0)
    xb = x_ref[...]
    h = (jnp.dot(mean, wl_ref[...], preferred_element_type=jnp.float32,
                 precision=lax.Precision.HIGHEST)
         + jnp.dot(xb, wr_ref[...], preferred_element_type=jnp.float32,
                   precision=lax.Precision.HIGHEST)
         + b_ref[...] + xb)
    h = jnp.maximum(h, 0.0)
    mu = jnp.mean(h, axis=-1, keepdims=True)
    var = jnp.mean((h - mu) ** 2, axis=-1, keepdims=True)
    o_ref[...] = ((h - mu) * lax.rsqrt(var + 1e-5) * g_ref[...]
                  + beta_ref[...])


_tc_fuse = pl.pallas_call(
    _tc_fuse_body,
    grid=(N // ROWS_PER_BLK,),
    in_specs=[
        pl.BlockSpec((NC, ROWS_PER_BLK, D), lambda i: (0, i, 0)),
        pl.BlockSpec((NC, ROWS_PER_BLK, CW), lambda i: (0, i, 0)),
        pl.BlockSpec((ROWS_PER_BLK, D), lambda i: (i, 0)),
        pl.BlockSpec((D, D), lambda i: (0, 0)),
        pl.BlockSpec((D, D), lambda i: (0, 0)),
        pl.BlockSpec((1, D), lambda i: (0, 0)),
        pl.BlockSpec((1, D), lambda i: (0, 0)),
        pl.BlockSpec((1, D), lambda i: (0, 0)),
    ],
    out_specs=pl.BlockSpec((ROWS_PER_BLK, D), lambda i: (i, 0)),
    out_shape=jax.ShapeDtypeStruct((N, D), jnp.float32),
)


@functools.partial(jax.jit, static_argnames=())
def kernel(x, edge_index, W_l, b_l, W_r, ln_gamma, ln_beta):
    src = edge_index[0].reshape(NW, NCHUNK, C)
    dst = edge_index[1].reshape(NW, NCHUNK, C)
    zrows = jnp.zeros((RPT, D), jnp.float32)
    consts = jnp.concatenate([jnp.zeros((RPT, CW), jnp.float32),
                              jnp.ones((C, CW), jnp.float32)], axis=0)
    part, cntp = _sc_aggregate(x, src, dst, zrows, consts)
    return _tc_fuse(part, cntp, x, W_l.T, W_r.T,
                    b_l.reshape(1, D), ln_gamma.reshape(1, D),
                    ln_beta.reshape(1, D))
